# Initial kernel scaffold; baseline (speedup 1.0000x reference)
import jax
import jax.numpy as jnp
from jax.experimental import pallas as pl


def kernel(z, W):
    raise NotImplementedError("write your pallas kernel here")


# speedup vs baseline: 1.1416x; 1.1416x over previous
import functools

import jax
import jax.numpy as jnp
from jax.experimental import pallas as pl

NUM_CODES = 1024
HIDDEN = 256
BETA = 0.25

_T = 512


def _vq_body(zf_ref, w_ref, zq_ref, idx_ref, loss_ref):
    i = pl.program_id(0)
    nsteps = pl.num_programs(0)

    w = w_ref[...]
    w2 = jnp.sum(w * w, axis=1)
    zf = zf_ref[...]
    zf2 = jnp.sum(zf * zf, axis=1, keepdims=True)

    s = jax.lax.dot_general(
        zf, w, (((1,), (1,)), ((), ())),
        preferred_element_type=jnp.float32)
    d = (zf2 + w2[None, :]) - 2.0 * s

    dmin = jnp.min(d, axis=1, keepdims=True)
    col = jax.lax.broadcasted_iota(jnp.int32, d.shape, 1)
    idx = jnp.min(jnp.where(d == dmin, col, NUM_CODES), axis=1)
    idx_ref[...] = idx.reshape(idx_ref.shape)

    onehot = (idx[:, None] == col).astype(jnp.float32)
    zq = jax.lax.dot_general(
        onehot, w, (((1,), (0,)), ((), ())),
        preferred_element_type=jnp.float32,
        precision=jax.lax.Precision.HIGHEST)
    zq_ref[...] = zf + (zq - zf)

    part = jnp.sum(dmin).reshape(1, 1)

    @pl.when(i == 0)
    def _init():
        loss_ref[...] = jnp.zeros_like(loss_ref)

    loss_ref[...] += part

    @pl.when(i == nsteps - 1)
    def _final():
        n_elems = nsteps * _T * HIDDEN
        loss_ref[...] = loss_ref[...] * ((1.0 + BETA) / n_elems)


@functools.partial(jax.jit, static_argnames=())
def kernel(z, W):
    B, C, H, Wsp = z.shape
    ntok = B * H * Wsp
    zf = jnp.transpose(z, (0, 2, 3, 1)).reshape(ntok, C)
    grid = ntok // _T

    zq, idx, loss = pl.pallas_call(
        _vq_body,
        grid=(grid,),
        in_specs=[
            pl.BlockSpec((_T, C), lambda i: (i, 0)),
            pl.BlockSpec((NUM_CODES, C), lambda i: (0, 0)),
        ],
        out_specs=[
            pl.BlockSpec((_T, C), lambda i: (i, 0)),
            pl.BlockSpec((1, 1, _T), lambda i: (i, 0, 0)),
            pl.BlockSpec((1, 1), lambda i: (0, 0)),
        ],
        out_shape=[
            jax.ShapeDtypeStruct((ntok, C), jnp.float32),
            jax.ShapeDtypeStruct((grid, 1, _T), jnp.int32),
            jax.ShapeDtypeStruct((1, 1), jnp.float32),
        ],
    )(zf, W)

    z_q = jnp.transpose(zq.reshape(B, H, Wsp, C), (0, 3, 1, 2))
    return (z_q, idx.reshape(ntok), loss[0, 0])
